# baseline (speedup 1.0000x reference)
import functools

import jax
import jax.numpy as jnp
from jax import lax
from jax.experimental import pallas as pl
from jax.experimental.pallas import tpu as pltpu
from jax.experimental.pallas import tpu_sc as plsc

N = 10000
E = 160000
G = 64
D_IN = 256
D_HID = 256
D_OUT = 128

NC = 2
NS = 16
CHUNK = 128
E_PAD = 163840
PCH = 40
NSLOT = 2
ACC_ROWS = 10112
NP16 = N + 16
EPS = 1e-5

_sc_mesh = plsc.VectorSubcoreMesh(core_axis_name="c", subcore_axis_name="s")


def _deg_body(dst_r, out, dbuf, counts):
    c = lax.axis_index("c")
    s = lax.axis_index("s")
    w = s * NC + c
    def _zero(i, _):
        counts[pl.ds(i * 16, 16)] = jnp.zeros((16,), jnp.float32)
        return 0
    lax.fori_loop(0, NP16 // 16, _zero, 0)
    pltpu.sync_copy(dst_r.at[w], dbuf)
    one = jnp.ones((16,), jnp.float32)

    def _count(a, _):
        for b in range(8):
            idx = dbuf[a, pl.ds(b * 16, 16)]
            plsc.addupdate_scatter(counts, [idx], one)
        return 0
    lax.fori_loop(0, 40, _count, 0)
    pltpu.sync_copy(counts, out.at[pl.ds(w * NP16, NP16)])


def _deg_partials(dst_r32):
    return pl.kernel(
        _deg_body,
        out_type=jax.ShapeDtypeStruct((NC * NS * NP16,), jnp.float32),
        mesh=_sc_mesh,
        compiler_params=pltpu.CompilerParams(needs_layout_passes=False),
        scratch_types=[
            pltpu.VMEM((40, 128), jnp.int32),
            pltpu.VMEM((NP16,), jnp.float32),
        ],
    )(dst_r32)


OUT_ROWS = ACC_ROWS


def _make_segsum_body(edge_split):
    npieces = 1 if edge_split else 2

    def body(ytab, idx_r, zrows, out, ibuf, rows, acc, gsem):
        c = lax.axis_index("c")
        s = lax.axis_index("s")
        zr = ACC_ROWS // NS
        pltpu.sync_copy(zrows, acc.at[pl.ds(s * zr, zr)])
        plsc.subcore_barrier()

        table = ytab.at[0] if edge_split else ytab.at[c]

        def gather_start(ps, j, p):
            pltpu.async_copy(table.at[ibuf.at[ps, 0, j]], rows.at[p],
                             gsem.at[p])

        def gather_wait(ps, j, p):
            pltpu.make_async_copy(table.at[ibuf.at[ps, 0, j]], rows.at[p],
                                  gsem.at[p]).wait()

        def scat_sync(ps, j, p):
            pltpu.sync_copy(rows.at[p], acc.at[ibuf.at[ps, 1, j]], add=True)

        for q in range(npieces):
            piece = c if edge_split else q
            pltpu.sync_copy(idx_r.at[s, piece], ibuf.at[0])

            def _step(g, _):
                p = lax.rem(g, NSLOT)

                @pl.when(g >= 2)
                def _():
                    gather_wait(0, g - 2, p)
                    scat_sync(0, g - 2, p)

                @pl.when(g < PCH)
                def _():
                    gather_start(0, g, p)
                return 0

            lax.fori_loop(0, PCH + 2, _step, 0)

        plsc.subcore_barrier()
        pltpu.sync_copy(acc.at[pl.ds(s * zr, zr)],
                        out.at[c, pl.ds(s * zr, zr)])
    return body


def _segsum(ytab, idx_r, zrows, edge_split):
    d2 = ytab.shape[2]
    return pl.kernel(
        _make_segsum_body(edge_split),
        out_type=jax.ShapeDtypeStruct((NC, OUT_ROWS, d2), jnp.float32),
        mesh=_sc_mesh,
        compiler_params=pltpu.CompilerParams(needs_layout_passes=False),
        scratch_types=[
            pltpu.VMEM((1, 2, PCH, CHUNK), jnp.int32),
            pltpu.VMEM((NSLOT, CHUNK, d2), jnp.float32),
            pltpu.VMEM_SHARED((ACC_ROWS, d2), jnp.float32),
            pltpu.SemaphoreType.DMA((NSLOT,)),
        ],
    )(ytab, idx_r, zrows)


DHALF = 5056
DACC = 5120
CROWS = 81


def _dsplit_body(ytab, idx_r, zrows, out, isrc, idst, cbuf, uidx, rows, acc, gsem):
    c = lax.axis_index("c")
    s = lax.axis_index("s")
    zr = DACC // NS
    pltpu.sync_copy(zrows, acc.at[pl.ds(s * zr, zr)])
    plsc.subcore_barrier()
    table = ytab.at[0]
    base = c * DHALF
    lanes = lax.iota(jnp.int32, 16)

    cnt = jnp.int32(0)
    for q in range(8):
        pltpu.sync_copy(idx_r.at[s, q, 0], isrc)
        pltpu.sync_copy(idx_r.at[s, q, 1], idst)

        def _scan(a, cnt):
            for b in range(8):
                srcv = isrc[a, pl.ds(b * 16, 16)]
                dstv = idst[a, pl.ds(b * 16, 16)]
                dl = dstv - base
                mask = (dl >= 0) & (dl < DHALF)
                incl = plsc.cumsum(mask.astype(jnp.int32))
                pos = cnt + incl - 1
                packed = srcv | lax.shift_left(dl, 16)
                plsc.store_scatter(
                    cbuf, [lax.shift_right_logical(pos, 7),
                           lax.bitwise_and(pos, 127)], packed, mask=mask)
                cnt = cnt + jnp.sum(mask.astype(jnp.int32))
            return cnt

        cnt = lax.fori_loop(0, 10, _scan, cnt)

    padval = jnp.full((16,), DHALF * 65536, jnp.int32)
    for k2 in range(8):
        pos = cnt + k2 * 16 + lanes
        plsc.store_scatter(
            cbuf, [lax.shift_right_logical(pos, 7),
                   lax.bitwise_and(pos, 127)], padval)
    nrows = lax.shift_right_logical(cnt + 128, 7)

    def unpack(r, h):
        for j in range(4):
            w = cbuf[r, pl.ds(h * 64 + j * 16, 16)]
            uidx[0, h, 0, pl.ds(j * 16, 16)] = lax.bitwise_and(w, 65535)
            uidx[1, h, 0, pl.ds(j * 16, 16)] = lax.shift_right_logical(w, 16)

    def gather_start(h):
        pltpu.async_copy(table.at[uidx.at[0, h, 0]], rows.at[h], gsem.at[h])

    def gather_wait(h):
        pltpu.make_async_copy(table.at[uidx.at[0, h, 0]], rows.at[h],
                              gsem.at[h]).wait()

    def scat_sync(h):
        pltpu.sync_copy(rows.at[h], acc.at[uidx.at[1, h, 0]], add=True)

    def _step(r, _):
        for h in range(2):
            @pl.when(r >= 1)
            def _():
                gather_wait(h)
                scat_sync(h)

            @pl.when(r < nrows)
            def _():
                unpack(r, h)
                gather_start(h)
        return 0

    lax.fori_loop(0, nrows + 1, _step, 0)
    plsc.subcore_barrier()
    pltpu.sync_copy(acc.at[pl.ds(s * zr, zr)], out.at[c, pl.ds(s * zr, zr)])


def _segsum_dsplit(ytab, idx_r, zrows):
    return pl.kernel(
        _dsplit_body,
        out_type=jax.ShapeDtypeStruct((NC, DACC, 2, 128), jnp.float32),
        mesh=_sc_mesh,
        compiler_params=pltpu.CompilerParams(needs_layout_passes=False),
        scratch_types=[
            pltpu.VMEM((10, 128), jnp.int32),
            pltpu.VMEM((10, 128), jnp.int32),
            pltpu.VMEM((CROWS, 128), jnp.int32),
            pltpu.VMEM((2, 2, 1, 64), jnp.int32),
            pltpu.VMEM((2, 64, 2, 128), jnp.float32),
            pltpu.VMEM_SHARED((DACC, 2, 128), jnp.float32),
            pltpu.SemaphoreType.DMA((2,)),
        ],
    )(ytab, idx_r, zrows)


_BN = 1000


def _y1_body(degp_ref, x_ref, w_ref, y_ref, dinv_ref):
    deg = jnp.sum(degp_ref[...], axis=1) + 1.0
    dinv = lax.rsqrt(deg)
    y = jnp.dot(x_ref[...], w_ref[...],
                preferred_element_type=jnp.float32) * dinv[:, None]
    y_ref[0] = y
    dinv_ref[...] = dinv[:, None]


def _y1_call(deg_parts, x, W1):
    return pl.pallas_call(
        _y1_body,
        grid=(N // _BN,),
        in_specs=[
            pl.BlockSpec((_BN, NC * NS), lambda i: (i, 0)),
            pl.BlockSpec((_BN, D_IN), lambda i: (i, 0)),
            pl.BlockSpec((D_IN, D_HID), lambda i: (0, 0)),
        ],
        out_specs=[
            pl.BlockSpec((1, _BN, D_HID), lambda i: (0, i, 0)),
            pl.BlockSpec((_BN, 1), lambda i: (i, 0)),
        ],
        out_shape=[
            jax.ShapeDtypeStruct((1, N, D_HID), jnp.float32),
            jax.ShapeDtypeStruct((N, 1), jnp.float32),
        ],
    )(deg_parts, x, W1)


def _agg(s_ref, y_ref, dinv_ref, b_ref):
    agg = y_ref[0]
    for i in range(s_ref.shape[0]):
        agg = agg + s_ref[i]
    return agg * dinv_ref[...] + b_ref[...]


_B1 = DHALF // 8


def _stats_l1_body(s_ref, y_ref, dinv_ref, b_ref, o_ref):
    co, i = pl.program_id(0), pl.program_id(1)
    agg = _agg(s_ref, y_ref, dinv_ref, b_ref)
    base = (co * 8 + i) * _B1
    riota = lax.broadcasted_iota(jnp.int32, (_B1, 1), 0)
    agg = jnp.where(base + riota < N, agg, 0.0)
    part = jnp.concatenate(
        [jnp.sum(agg, axis=0)[None], jnp.sum(agg * agg, axis=0)[None]],
        axis=0)

    @pl.when((co == 0) & (i == 0))
    def _():
        o_ref[...] = jnp.zeros_like(o_ref)

    o_ref[...] += part


def _stats_l1_call(S1, y1, dinv, b):
    d = b.shape[1]
    return pl.pallas_call(
        _stats_l1_body,
        grid=(NC, 8),
        in_specs=[
            pl.BlockSpec((1, _B1, d), lambda co, i: (co, i, 0)),
            pl.BlockSpec((1, _B1, d), lambda co, i: (0, co * 8 + i, 0)),
            pl.BlockSpec((_B1, 1), lambda co, i: (co * 8 + i, 0)),
            pl.BlockSpec((1, d), lambda co, i: (0, 0)),
        ],
        out_specs=pl.BlockSpec((2, d), lambda co, i: (0, 0)),
        out_shape=jax.ShapeDtypeStruct((2, d), jnp.float32),
    )(S1, y1, dinv, b)


def _stats_l2_call(S, y, dinv, b):
    d = b.shape[1]

    def body(s_ref, y_ref, dinv_ref, b_ref, o_ref):
        agg = _agg(s_ref, y_ref, dinv_ref, b_ref)
        part = jnp.concatenate(
            [jnp.sum(agg, axis=0)[None], jnp.sum(agg * agg, axis=0)[None]],
            axis=0)

        @pl.when(pl.program_id(0) == 0)
        def _():
            o_ref[...] = jnp.zeros_like(o_ref)

        o_ref[...] += part

    return pl.pallas_call(
        body,
        grid=(N // _BN,),
        in_specs=[
            pl.BlockSpec((S.shape[0], _BN, d), lambda i: (0, i, 0)),
            pl.BlockSpec((1, _BN, d), lambda i: (0, i, 0)),
            pl.BlockSpec((_BN, 1), lambda i: (i, 0)),
            pl.BlockSpec((1, d), lambda i: (0, 0)),
        ],
        out_specs=pl.BlockSpec((2, d), lambda i: (0, 0)),
        out_shape=jax.ShapeDtypeStruct((2, d), jnp.float32),
    )(S, y, dinv, b)


def _bn_relu_from_stats(agg, stats_ref, gamma_ref, beta_ref):
    mu = stats_ref[0][None] / N
    var = stats_ref[1][None] / N - mu * mu
    rstd = lax.rsqrt(var + EPS)
    return jnp.maximum((agg - mu) * rstd * gamma_ref[...] + beta_ref[...], 0.0)


def _y2_body(s_ref, y_ref, dinv_ref, b_ref, stats_ref, gamma_ref, beta_ref,
             w_ref, y2_ref):
    agg = _agg(s_ref, y_ref, dinv_ref, b_ref)
    h = _bn_relu_from_stats(agg, stats_ref, gamma_ref, beta_ref)
    y2 = jnp.dot(h, w_ref[...],
                 preferred_element_type=jnp.float32) * dinv_ref[...]
    y2_ref[0] = y2


def _y2_call(S1, y1, dinv, b1, stats1, gamma1, beta1, W2):
    return pl.pallas_call(
        _y2_body,
        grid=(NC, 8),
        in_specs=[
            pl.BlockSpec((1, _B1, D_HID), lambda co, i: (co, i, 0)),
            pl.BlockSpec((1, _B1, D_HID), lambda co, i: (0, co * 8 + i, 0)),
            pl.BlockSpec((_B1, 1), lambda co, i: (co * 8 + i, 0)),
            pl.BlockSpec((1, D_HID), lambda co, i: (0, 0)),
            pl.BlockSpec((2, D_HID), lambda co, i: (0, 0)),
            pl.BlockSpec((1, D_HID), lambda co, i: (0, 0)),
            pl.BlockSpec((1, D_HID), lambda co, i: (0, 0)),
            pl.BlockSpec((D_HID, D_OUT), lambda co, i: (0, 0)),
        ],
        out_specs=pl.BlockSpec((1, _B1, D_OUT), lambda co, i: (0, co * 8 + i, 0)),
        out_shape=jax.ShapeDtypeStruct((1, NC * DHALF, D_OUT), jnp.float32),
    )(S1, y1, dinv, b1, stats1, gamma1, beta1, W2)


def _pool_body(s_ref, y_ref, dinv_ref, b_ref, stats_ref, gamma_ref, beta_ref,
               batch_ref, o_ref, pool_acc, cnt_acc):
    i = pl.program_id(0)
    agg = _agg(s_ref, y_ref, dinv_ref, b_ref)
    h = _bn_relu_from_stats(agg, stats_ref, gamma_ref, beta_ref)
    gids = lax.broadcasted_iota(jnp.int32, (_BN, G), 1)
    onehot = (batch_ref[...] == gids).astype(jnp.float32)

    @pl.when(i == 0)
    def _():
        pool_acc[...] = jnp.zeros_like(pool_acc)
        cnt_acc[...] = jnp.zeros_like(cnt_acc)

    dn = (((0,), (0,)), ((), ()))
    pool_acc[...] += lax.dot_general(onehot, h, dn,
                                     preferred_element_type=jnp.float32)
    cnt_acc[...] += lax.dot_general(onehot, jnp.ones_like(h), dn,
                                    preferred_element_type=jnp.float32)

    @pl.when(i == pl.num_programs(0) - 1)
    def _():
        o_ref[...] = pool_acc[...] / jnp.maximum(cnt_acc[...], 1.0)


def _pool_call(S2, y2, dinv, b2, stats2, gamma2, beta2, batch2):
    return pl.pallas_call(
        _pool_body,
        grid=(N // _BN,),
        in_specs=[
            pl.BlockSpec((NC, _BN, D_OUT), lambda i: (0, i, 0)),
            pl.BlockSpec((1, _BN, D_OUT), lambda i: (0, i, 0)),
            pl.BlockSpec((_BN, 1), lambda i: (i, 0)),
            pl.BlockSpec((1, D_OUT), lambda i: (0, 0)),
            pl.BlockSpec((2, D_OUT), lambda i: (0, 0)),
            pl.BlockSpec((1, D_OUT), lambda i: (0, 0)),
            pl.BlockSpec((1, D_OUT), lambda i: (0, 0)),
            pl.BlockSpec((_BN, 1), lambda i: (i, 0)),
        ],
        out_specs=pl.BlockSpec((G, D_OUT), lambda i: (0, 0)),
        out_shape=jax.ShapeDtypeStruct((G, D_OUT), jnp.float32),
        scratch_shapes=[
            pltpu.VMEM((G, D_OUT), jnp.float32),
            pltpu.VMEM((G, D_OUT), jnp.float32),
        ],
    )(S2, y2, dinv, b2, stats2, gamma2, beta2, batch2)


def kernel(x, edge_index, batch, W1, b1, gamma1, beta1, W2, b2, gamma2, beta2):
    src = edge_index[0]
    dst = edge_index[1]
    padn = E_PAD - E
    srcp = jnp.concatenate([src, jnp.zeros((padn,), jnp.int32)])
    dstp = jnp.concatenate([dst, jnp.full((padn,), N, jnp.int32)])
    idx_r = jnp.stack([srcp.reshape(NS, 2, PCH, CHUNK),
                       dstp.reshape(NS, 2, PCH, CHUNK)], axis=2)
    idx_r4 = jnp.stack([srcp.reshape(NS, 8, 10, CHUNK),
                        dstp.reshape(NS, 8, 10, CHUNK)], axis=2)
    dst_r32 = dstp.reshape(NC * NS, 40, 128)
    z128 = jnp.zeros((ACC_ROWS // NS, 128), jnp.float32)
    z256 = jnp.zeros((DACC // NS, 2, 128), jnp.float32)
    batch2 = batch[:, None]
    b1r, g1r, be1r = b1[None], gamma1[None], beta1[None]
    b2r, g2r, be2r = b2[None], gamma2[None], beta2[None]

    deg_parts = _deg_partials(dst_r32).reshape(NC * NS, NP16)
    deg_t = jnp.transpose(deg_parts)[:N]
    y1, dinv = _y1_call(deg_t, x, W1)
    S1 = _segsum_dsplit(y1.reshape(1, N, 2, 128), idx_r4,
                        z256).reshape(NC, DACC, D_HID)
    stats1 = _stats_l1_call(S1, y1, dinv, b1r)
    y2 = _y2_call(S1, y1, dinv, b1r, stats1, g1r, be1r, W2)
    S2 = _segsum(y2, idx_r, z128, edge_split=True)
    stats2 = _stats_l2_call(S2, y2, dinv, b2r)
    return _pool_call(S2, y2, dinv, b2r, stats2, g2r, be2r, batch2)


# speedup vs baseline: 1.4166x; 1.4166x over previous
import functools

import jax
import jax.numpy as jnp
from jax import lax
from jax.experimental import pallas as pl
from jax.experimental.pallas import tpu as pltpu
from jax.experimental.pallas import tpu_sc as plsc

N = 10000
E = 160000
G = 64
D_IN = 256
D_HID = 256
D_OUT = 128

NC = 2
NS = 16
CHUNK = 128
E_PAD = 163840
PCH = 40
NSLOT = 2
ACC_ROWS = 10112
NP16 = N + 16
EPS = 1e-5

_sc_mesh = plsc.VectorSubcoreMesh(core_axis_name="c", subcore_axis_name="s")


def _deg_body(dst_r, out, dbuf, counts):
    c = lax.axis_index("c")
    s = lax.axis_index("s")
    w = s * NC + c
    def _zero(i, _):
        counts[pl.ds(i * 16, 16)] = jnp.zeros((16,), jnp.float32)
        return 0
    lax.fori_loop(0, NP16 // 16, _zero, 0)
    pltpu.sync_copy(dst_r.at[w], dbuf)
    one = jnp.ones((16,), jnp.float32)

    def _count(a, _):
        for b in range(8):
            idx = dbuf[a, pl.ds(b * 16, 16)]
            plsc.addupdate_scatter(counts, [idx], one)
        return 0
    lax.fori_loop(0, 40, _count, 0)
    pltpu.sync_copy(counts, out.at[pl.ds(w * NP16, NP16)])


def _deg_partials(dst_r32):
    return pl.kernel(
        _deg_body,
        out_type=jax.ShapeDtypeStruct((NC * NS * NP16,), jnp.float32),
        mesh=_sc_mesh,
        compiler_params=pltpu.CompilerParams(needs_layout_passes=False),
        scratch_types=[
            pltpu.VMEM((40, 128), jnp.int32),
            pltpu.VMEM((NP16,), jnp.float32),
        ],
    )(dst_r32)


OUT_ROWS = ACC_ROWS


def _make_segsum_body(edge_split):
    npieces = 1 if edge_split else 2

    def body(ytab, idx_r, zrows, out, ibuf, rows, acc, gsem):
        c = lax.axis_index("c")
        s = lax.axis_index("s")
        zr = ACC_ROWS // NS
        pltpu.sync_copy(zrows, acc.at[pl.ds(s * zr, zr)])
        plsc.subcore_barrier()

        table = ytab.at[0] if edge_split else ytab.at[c]

        def gather_start(ps, j, p):
            pltpu.async_copy(table.at[ibuf.at[ps, 0, j]], rows.at[p],
                             gsem.at[p])

        def gather_wait(ps, j, p):
            pltpu.make_async_copy(table.at[ibuf.at[ps, 0, j]], rows.at[p],
                                  gsem.at[p]).wait()

        def scat_sync(ps, j, p):
            pltpu.sync_copy(rows.at[p], acc.at[ibuf.at[ps, 1, j]], add=True)

        for q in range(npieces):
            piece = c if edge_split else q
            pltpu.sync_copy(idx_r.at[s, piece], ibuf.at[0])

            def _step(g2, _):
                for h in range(2):
                    @pl.when(g2 >= 1)
                    def _():
                        gather_wait(0, 2 * g2 - 2 + h, h)
                        scat_sync(0, 2 * g2 - 2 + h, h)

                    @pl.when(g2 < PCH // 2)
                    def _():
                        gather_start(0, 2 * g2 + h, h)
                return 0

            lax.fori_loop(0, PCH // 2 + 1, _step, 0)

        plsc.subcore_barrier()
        pltpu.sync_copy(acc.at[pl.ds(s * zr, zr)],
                        out.at[c, pl.ds(s * zr, zr)])
    return body


def _segsum(ytab, idx_r, zrows, edge_split):
    d2 = ytab.shape[2]
    return pl.kernel(
        _make_segsum_body(edge_split),
        out_type=jax.ShapeDtypeStruct((NC, OUT_ROWS, d2), jnp.float32),
        mesh=_sc_mesh,
        compiler_params=pltpu.CompilerParams(needs_layout_passes=False),
        scratch_types=[
            pltpu.VMEM((1, 2, PCH, CHUNK), jnp.int32),
            pltpu.VMEM((NSLOT, CHUNK, d2), jnp.float32),
            pltpu.VMEM_SHARED((ACC_ROWS, d2), jnp.float32),
            pltpu.SemaphoreType.DMA((NSLOT,)),
        ],
    )(ytab, idx_r, zrows)


_BN = 1000


def _y1_body(degp_ref, x_ref, w_ref, y_ref, dinv_ref):
    deg = jnp.sum(degp_ref[...], axis=1) + 1.0
    dinv = lax.rsqrt(deg)
    y = jnp.dot(x_ref[...], w_ref[...],
                preferred_element_type=jnp.float32) * dinv[:, None]
    d2 = y.shape[1] // 2
    y_ref[0] = y[:, :d2]
    y_ref[1] = y[:, d2:]
    dinv_ref[...] = dinv[:, None]


def _y1_call(deg_parts, x, W1):
    return pl.pallas_call(
        _y1_body,
        grid=(N // _BN,),
        in_specs=[
            pl.BlockSpec((_BN, NC * NS), lambda i: (i, 0)),
            pl.BlockSpec((_BN, D_IN), lambda i: (i, 0)),
            pl.BlockSpec((D_IN, D_HID), lambda i: (0, 0)),
        ],
        out_specs=[
            pl.BlockSpec((NC, _BN, D_HID // 2), lambda i: (0, i, 0)),
            pl.BlockSpec((_BN, 1), lambda i: (i, 0)),
        ],
        out_shape=[
            jax.ShapeDtypeStruct((NC, N, D_HID // 2), jnp.float32),
            jax.ShapeDtypeStruct((N, 1), jnp.float32),
        ],
    )(deg_parts, x, W1)


def _agg(s_ref, y_ref, dinv_ref, b_ref, col_split):
    dinv = dinv_ref[...]
    if col_split:
        agg = jnp.concatenate(
            [(s_ref[0] + y_ref[0]), (s_ref[1] + y_ref[1])], axis=1)
    else:
        agg = s_ref[0] + s_ref[1] + y_ref[0]
    return agg * dinv + b_ref[...]


def _make_stats_body(col_split):
    def body(s_ref, y_ref, dinv_ref, b_ref, o_ref):
        agg = _agg(s_ref, y_ref, dinv_ref, b_ref, col_split)
        part = jnp.concatenate(
            [jnp.sum(agg, axis=0)[None], jnp.sum(agg * agg, axis=0)[None]],
            axis=0)

        @pl.when(pl.program_id(0) == 0)
        def _():
            o_ref[...] = jnp.zeros_like(o_ref)

        o_ref[...] += part
    return body


def _stats_call(S, y, dinv, b, col_split):
    d = b.shape[1]
    return pl.pallas_call(
        _make_stats_body(col_split),
        grid=(N // _BN,),
        in_specs=[
            pl.BlockSpec((S.shape[0], _BN, S.shape[2]), lambda i: (0, i, 0)),
            pl.BlockSpec((y.shape[0], _BN, y.shape[2]), lambda i: (0, i, 0)),
            pl.BlockSpec((_BN, 1), lambda i: (i, 0)),
            pl.BlockSpec((1, d), lambda i: (0, 0)),
        ],
        out_specs=pl.BlockSpec((2, d), lambda i: (0, 0)),
        out_shape=jax.ShapeDtypeStruct((2, d), jnp.float32),
    )(S, y, dinv, b)


def _bn_relu_from_stats(agg, stats_ref, gamma_ref, beta_ref):
    mu = stats_ref[0][None] / N
    var = stats_ref[1][None] / N - mu * mu
    rstd = lax.rsqrt(var + EPS)
    return jnp.maximum((agg - mu) * rstd * gamma_ref[...] + beta_ref[...], 0.0)


def _y2_body(s_ref, y_ref, dinv_ref, b_ref, stats_ref, gamma_ref, beta_ref,
             w_ref, y2_ref):
    agg = _agg(s_ref, y_ref, dinv_ref, b_ref, col_split=True)
    h = _bn_relu_from_stats(agg, stats_ref, gamma_ref, beta_ref)
    y2 = jnp.dot(h, w_ref[...],
                 preferred_element_type=jnp.float32) * dinv_ref[...]
    y2_ref[0] = y2


def _y2_call(S1, y1, dinv, b1, stats1, gamma1, beta1, W2):
    return pl.pallas_call(
        _y2_body,
        grid=(N // _BN,),
        in_specs=[
            pl.BlockSpec((NC, _BN, D_HID // 2), lambda i: (0, i, 0)),
            pl.BlockSpec((NC, _BN, D_HID // 2), lambda i: (0, i, 0)),
            pl.BlockSpec((_BN, 1), lambda i: (i, 0)),
            pl.BlockSpec((1, D_HID), lambda i: (0, 0)),
            pl.BlockSpec((2, D_HID), lambda i: (0, 0)),
            pl.BlockSpec((1, D_HID), lambda i: (0, 0)),
            pl.BlockSpec((1, D_HID), lambda i: (0, 0)),
            pl.BlockSpec((D_HID, D_OUT), lambda i: (0, 0)),
        ],
        out_specs=pl.BlockSpec((1, _BN, D_OUT), lambda i: (0, i, 0)),
        out_shape=jax.ShapeDtypeStruct((1, N, D_OUT), jnp.float32),
    )(S1, y1, dinv, b1, stats1, gamma1, beta1, W2)


def _pool_body(s_ref, y_ref, dinv_ref, b_ref, stats_ref, gamma_ref, beta_ref,
               batch_ref, o_ref, pool_acc, cnt_acc):
    i = pl.program_id(0)
    agg = _agg(s_ref, y_ref, dinv_ref, b_ref, col_split=False)
    h = _bn_relu_from_stats(agg, stats_ref, gamma_ref, beta_ref)
    gids = lax.broadcasted_iota(jnp.int32, (_BN, G), 1)
    onehot = (batch_ref[...] == gids).astype(jnp.float32)

    @pl.when(i == 0)
    def _():
        pool_acc[...] = jnp.zeros_like(pool_acc)
        cnt_acc[...] = jnp.zeros_like(cnt_acc)

    dn = (((0,), (0,)), ((), ()))
    pool_acc[...] += lax.dot_general(onehot, h, dn,
                                     preferred_element_type=jnp.float32)
    cnt_acc[...] += lax.dot_general(onehot, jnp.ones_like(h), dn,
                                    preferred_element_type=jnp.float32)

    @pl.when(i == pl.num_programs(0) - 1)
    def _():
        o_ref[...] = pool_acc[...] / jnp.maximum(cnt_acc[...], 1.0)


def _pool_call(S2, y2, dinv, b2, stats2, gamma2, beta2, batch2):
    return pl.pallas_call(
        _pool_body,
        grid=(N // _BN,),
        in_specs=[
            pl.BlockSpec((NC, _BN, D_OUT), lambda i: (0, i, 0)),
            pl.BlockSpec((1, _BN, D_OUT), lambda i: (0, i, 0)),
            pl.BlockSpec((_BN, 1), lambda i: (i, 0)),
            pl.BlockSpec((1, D_OUT), lambda i: (0, 0)),
            pl.BlockSpec((2, D_OUT), lambda i: (0, 0)),
            pl.BlockSpec((1, D_OUT), lambda i: (0, 0)),
            pl.BlockSpec((1, D_OUT), lambda i: (0, 0)),
            pl.BlockSpec((_BN, 1), lambda i: (i, 0)),
        ],
        out_specs=pl.BlockSpec((G, D_OUT), lambda i: (0, 0)),
        out_shape=jax.ShapeDtypeStruct((G, D_OUT), jnp.float32),
        scratch_shapes=[
            pltpu.VMEM((G, D_OUT), jnp.float32),
            pltpu.VMEM((G, D_OUT), jnp.float32),
        ],
    )(S2, y2, dinv, b2, stats2, gamma2, beta2, batch2)


def kernel(x, edge_index, batch, W1, b1, gamma1, beta1, W2, b2, gamma2, beta2):
    src = edge_index[0]
    dst = edge_index[1]
    padn = E_PAD - E
    srcp = jnp.concatenate([src, jnp.zeros((padn,), jnp.int32)])
    dstp = jnp.concatenate([dst, jnp.full((padn,), N, jnp.int32)])
    idx_r = jnp.stack([srcp.reshape(NS, 2, PCH, CHUNK),
                       dstp.reshape(NS, 2, PCH, CHUNK)], axis=2)
    dst_r32 = dstp.reshape(NC * NS, 40, 128)
    z128 = jnp.zeros((ACC_ROWS // NS, 128), jnp.float32)
    batch2 = batch[:, None]
    b1r, g1r, be1r = b1[None], gamma1[None], beta1[None]
    b2r, g2r, be2r = b2[None], gamma2[None], beta2[None]

    deg_parts = _deg_partials(dst_r32).reshape(NC * NS, NP16)
    deg_t = jnp.transpose(deg_parts)[:N]
    y1, dinv = _y1_call(deg_t, x, W1)
    S1 = _segsum(y1, idx_r, z128, edge_split=False)
    stats1 = _stats_call(S1, y1, dinv, b1r, col_split=True)
    y2 = _y2_call(S1, y1, dinv, b1r, stats1, g1r, be1r, W2)
    S2 = _segsum(y2, idx_r, z128, edge_split=True)
    stats2 = _stats_call(S2, y2, dinv, b2r, col_split=False)
    return _pool_call(S2, y2, dinv, b2r, stats2, g2r, be2r, batch2)


# speedup vs baseline: 1.4171x; 1.0003x over previous
import jax
import jax.numpy as jnp
from jax import lax
from jax.experimental import pallas as pl
from jax.experimental.pallas import tpu as pltpu
from jax.experimental.pallas import tpu_sc as plsc

N = 10000
E = 160000
G = 64
D_IN = 256
D_HID = 256
D_OUT = 128

NC = 2
NS = 16
CHUNK = 128
E_PAD = 163840
PCH = 40
NSLOT = 2
ACC_ROWS = 10112
NP16 = N + 16
EPS = 1e-5

_sc_mesh = plsc.VectorSubcoreMesh(core_axis_name="c", subcore_axis_name="s")


def _deg_body(dst_r, out, dbuf, counts):
    c = lax.axis_index("c")
    s = lax.axis_index("s")
    w = s * NC + c
    def _zero(i, _):
        counts[pl.ds(i * 16, 16)] = jnp.zeros((16,), jnp.float32)
        return 0
    lax.fori_loop(0, NP16 // 16, _zero, 0)
    pltpu.sync_copy(dst_r.at[w], dbuf)
    one = jnp.ones((16,), jnp.float32)

    def _count(a, _):
        for b in range(8):
            idx = dbuf[a, pl.ds(b * 16, 16)]
            plsc.addupdate_scatter(counts, [idx], one)
        return 0
    lax.fori_loop(0, 40, _count, 0)
    pltpu.sync_copy(counts, out.at[pl.ds(w * NP16, NP16)])


def _deg_partials(dst_r32):
    return pl.kernel(
        _deg_body,
        out_type=jax.ShapeDtypeStruct((NC * NS * NP16,), jnp.float32),
        mesh=_sc_mesh,
        compiler_params=pltpu.CompilerParams(needs_layout_passes=False),
        scratch_types=[
            pltpu.VMEM((40, 128), jnp.int32),
            pltpu.VMEM((NP16,), jnp.float32),
        ],
    )(dst_r32)


OUT_ROWS = ACC_ROWS


def _make_segsum_body(edge_split):
    npieces = 1 if edge_split else 2

    def body(ytab, idx_r, zrows, out, ibuf, rows, acc, gsem):
        c = lax.axis_index("c")
        s = lax.axis_index("s")
        zr = ACC_ROWS // NS
        pltpu.sync_copy(zrows, acc.at[pl.ds(s * zr, zr)])
        plsc.subcore_barrier()

        table = ytab.at[0] if edge_split else ytab.at[c]

        def gather_start(ps, j, p):
            pltpu.async_copy(table.at[ibuf.at[ps, 0, j]], rows.at[p],
                             gsem.at[p])

        def gather_wait(ps, j, p):
            pltpu.make_async_copy(table.at[ibuf.at[ps, 0, j]], rows.at[p],
                                  gsem.at[p]).wait()

        def scat_sync(ps, j, p):
            pltpu.sync_copy(rows.at[p], acc.at[ibuf.at[ps, 1, j]], add=True)

        for q in range(npieces):
            piece = c if edge_split else q
            pltpu.sync_copy(idx_r.at[s, piece], ibuf.at[0])

            def _step(g2, _):
                for h in range(2):
                    @pl.when(g2 >= 1)
                    def _():
                        gather_wait(0, 2 * g2 - 2 + h, h)
                        scat_sync(0, 2 * g2 - 2 + h, h)

                    @pl.when(g2 < PCH // 2)
                    def _():
                        gather_start(0, 2 * g2 + h, h)
                return 0

            lax.fori_loop(0, PCH // 2 + 1, _step, 0)

        plsc.subcore_barrier()
        pltpu.sync_copy(acc.at[pl.ds(s * zr, zr)],
                        out.at[c, pl.ds(s * zr, zr)])
    return body


def _segsum(ytab, idx_r, zrows, edge_split):
    d2 = ytab.shape[2]
    return pl.kernel(
        _make_segsum_body(edge_split),
        out_type=jax.ShapeDtypeStruct((NC, OUT_ROWS, d2), jnp.float32),
        mesh=_sc_mesh,
        compiler_params=pltpu.CompilerParams(needs_layout_passes=False),
        scratch_types=[
            pltpu.VMEM((1, 2, PCH, CHUNK), jnp.int32),
            pltpu.VMEM((NSLOT, CHUNK, d2), jnp.float32),
            pltpu.VMEM_SHARED((ACC_ROWS, d2), jnp.float32),
            pltpu.SemaphoreType.DMA((NSLOT,)),
        ],
    )(ytab, idx_r, zrows)


_BN = 1000


def _y1_body(degp_ref, x_ref, w_ref, y_ref, dinv_ref):
    deg = jnp.sum(degp_ref[...], axis=1) + 1.0
    dinv = lax.rsqrt(deg)
    y = jnp.dot(x_ref[...], w_ref[...],
                preferred_element_type=jnp.float32) * dinv[:, None]
    d2 = y.shape[1] // 2
    y_ref[0] = y[:, :d2]
    y_ref[1] = y[:, d2:]
    dinv_ref[...] = dinv[:, None]


def _y1_call(deg_parts, x, W1):
    return pl.pallas_call(
        _y1_body,
        grid=(N // _BN,),
        in_specs=[
            pl.BlockSpec((_BN, NC * NS), lambda i: (i, 0)),
            pl.BlockSpec((_BN, D_IN), lambda i: (i, 0)),
            pl.BlockSpec((D_IN, D_HID), lambda i: (0, 0)),
        ],
        out_specs=[
            pl.BlockSpec((NC, _BN, D_HID // 2), lambda i: (0, i, 0)),
            pl.BlockSpec((_BN, 1), lambda i: (i, 0)),
        ],
        out_shape=[
            jax.ShapeDtypeStruct((NC, N, D_HID // 2), jnp.float32),
            jax.ShapeDtypeStruct((N, 1), jnp.float32),
        ],
    )(deg_parts, x, W1)


def _agg(s_ref, y_ref, dinv_ref, b_ref, col_split):
    dinv = dinv_ref[...]
    if col_split:
        agg = jnp.concatenate(
            [(s_ref[0] + y_ref[0]), (s_ref[1] + y_ref[1])], axis=1)
    else:
        agg = s_ref[0] + s_ref[1] + y_ref[0]
    return agg * dinv + b_ref[...]


def _make_stats_body(col_split):
    def body(s_ref, y_ref, dinv_ref, b_ref, o_ref):
        agg = _agg(s_ref, y_ref, dinv_ref, b_ref, col_split)
        part = jnp.concatenate(
            [jnp.sum(agg, axis=0)[None], jnp.sum(agg * agg, axis=0)[None]],
            axis=0)

        @pl.when(pl.program_id(0) == 0)
        def _():
            o_ref[...] = jnp.zeros_like(o_ref)

        o_ref[...] += part
    return body


def _stats_call(S, y, dinv, b, col_split):
    d = b.shape[1]
    return pl.pallas_call(
        _make_stats_body(col_split),
        grid=(N // _BN,),
        in_specs=[
            pl.BlockSpec((S.shape[0], _BN, S.shape[2]), lambda i: (0, i, 0)),
            pl.BlockSpec((y.shape[0], _BN, y.shape[2]), lambda i: (0, i, 0)),
            pl.BlockSpec((_BN, 1), lambda i: (i, 0)),
            pl.BlockSpec((1, d), lambda i: (0, 0)),
        ],
        out_specs=pl.BlockSpec((2, d), lambda i: (0, 0)),
        out_shape=jax.ShapeDtypeStruct((2, d), jnp.float32),
    )(S, y, dinv, b)


def _bn_relu_from_stats(agg, stats_ref, gamma_ref, beta_ref):
    mu = stats_ref[0][None] / N
    var = stats_ref[1][None] / N - mu * mu
    rstd = lax.rsqrt(var + EPS)
    return jnp.maximum((agg - mu) * rstd * gamma_ref[...] + beta_ref[...], 0.0)


def _y2_body(s_ref, y_ref, dinv_ref, b_ref, stats_ref, gamma_ref, beta_ref,
             w_ref, y2_ref):
    agg = _agg(s_ref, y_ref, dinv_ref, b_ref, col_split=True)
    h = _bn_relu_from_stats(agg, stats_ref, gamma_ref, beta_ref)
    y2 = jnp.dot(h, w_ref[...],
                 preferred_element_type=jnp.float32) * dinv_ref[...]
    y2_ref[0] = y2


def _y2_call(S1, y1, dinv, b1, stats1, gamma1, beta1, W2):
    return pl.pallas_call(
        _y2_body,
        grid=(N // _BN,),
        in_specs=[
            pl.BlockSpec((NC, _BN, D_HID // 2), lambda i: (0, i, 0)),
            pl.BlockSpec((NC, _BN, D_HID // 2), lambda i: (0, i, 0)),
            pl.BlockSpec((_BN, 1), lambda i: (i, 0)),
            pl.BlockSpec((1, D_HID), lambda i: (0, 0)),
            pl.BlockSpec((2, D_HID), lambda i: (0, 0)),
            pl.BlockSpec((1, D_HID), lambda i: (0, 0)),
            pl.BlockSpec((1, D_HID), lambda i: (0, 0)),
            pl.BlockSpec((D_HID, D_OUT), lambda i: (0, 0)),
        ],
        out_specs=pl.BlockSpec((1, _BN, D_OUT), lambda i: (0, i, 0)),
        out_shape=jax.ShapeDtypeStruct((1, N, D_OUT), jnp.float32),
    )(S1, y1, dinv, b1, stats1, gamma1, beta1, W2)


def _pool_body(s_ref, y_ref, dinv_ref, b_ref, stats_ref, gamma_ref, beta_ref,
               batch_ref, o_ref, pool_acc, cnt_acc):
    i = pl.program_id(0)
    agg = _agg(s_ref, y_ref, dinv_ref, b_ref, col_split=False)
    h = _bn_relu_from_stats(agg, stats_ref, gamma_ref, beta_ref)
    gids = lax.broadcasted_iota(jnp.int32, (_BN, G), 1)
    onehot = (batch_ref[...] == gids).astype(jnp.float32)

    @pl.when(i == 0)
    def _():
        pool_acc[...] = jnp.zeros_like(pool_acc)
        cnt_acc[...] = jnp.zeros_like(cnt_acc)

    dn = (((0,), (0,)), ((), ()))
    pool_acc[...] += lax.dot_general(onehot, h, dn,
                                     preferred_element_type=jnp.float32)
    cnt_acc[...] += lax.dot_general(onehot, jnp.ones_like(h), dn,
                                    preferred_element_type=jnp.float32)

    @pl.when(i == pl.num_programs(0) - 1)
    def _():
        o_ref[...] = pool_acc[...] / jnp.maximum(cnt_acc[...], 1.0)


def _pool_call(S2, y2, dinv, b2, stats2, gamma2, beta2, batch2):
    return pl.pallas_call(
        _pool_body,
        grid=(N // _BN,),
        in_specs=[
            pl.BlockSpec((NC, _BN, D_OUT), lambda i: (0, i, 0)),
            pl.BlockSpec((1, _BN, D_OUT), lambda i: (0, i, 0)),
            pl.BlockSpec((_BN, 1), lambda i: (i, 0)),
            pl.BlockSpec((1, D_OUT), lambda i: (0, 0)),
            pl.BlockSpec((2, D_OUT), lambda i: (0, 0)),
            pl.BlockSpec((1, D_OUT), lambda i: (0, 0)),
            pl.BlockSpec((1, D_OUT), lambda i: (0, 0)),
            pl.BlockSpec((_BN, 1), lambda i: (i, 0)),
        ],
        out_specs=pl.BlockSpec((G, D_OUT), lambda i: (0, 0)),
        out_shape=jax.ShapeDtypeStruct((G, D_OUT), jnp.float32),
        scratch_shapes=[
            pltpu.VMEM((G, D_OUT), jnp.float32),
            pltpu.VMEM((G, D_OUT), jnp.float32),
        ],
    )(S2, y2, dinv, b2, stats2, gamma2, beta2, batch2)


def kernel(x, edge_index, batch, W1, b1, gamma1, beta1, W2, b2, gamma2, beta2):
    src = edge_index[0]
    dst = edge_index[1]
    padn = E_PAD - E
    srcp = jnp.concatenate([src, jnp.zeros((padn,), jnp.int32)])
    dstp = jnp.concatenate([dst, jnp.full((padn,), N, jnp.int32)])
    idx_r = jnp.stack([srcp.reshape(NS, 2, PCH, CHUNK),
                       dstp.reshape(NS, 2, PCH, CHUNK)], axis=2)
    dst_r32 = dstp.reshape(NC * NS, 40, 128)
    z128 = jnp.zeros((ACC_ROWS // NS, 128), jnp.float32)
    batch2 = batch[:, None]
    b1r, g1r, be1r = b1[None], gamma1[None], beta1[None]
    b2r, g2r, be2r = b2[None], gamma2[None], beta2[None]

    deg_parts = _deg_partials(dst_r32).reshape(NC * NS, NP16)
    deg_t = jnp.transpose(deg_parts)[:N]
    y1, dinv = _y1_call(deg_t, x, W1)
    S1 = _segsum(y1, idx_r, z128, edge_split=False)
    stats1 = _stats_call(S1, y1, dinv, b1r, col_split=True)
    y2 = _y2_call(S1, y1, dinv, b1r, stats1, g1r, be1r, W2)
    S2 = _segsum(y2, idx_r, z128, edge_split=True)
    stats2 = _stats_call(S2, y2, dinv, b2r, col_split=False)
    return _pool_call(S2, y2, dinv, b2r, stats2, g2r, be2r, batch2)
